# baseline (speedup 1.0000x reference)
import functools

import jax
import jax.numpy as jnp
from jax import lax
from jax.experimental import pallas as pl
from jax.experimental.pallas import tpu as pltpu
from jax.experimental.pallas import tpu_sc as plsc

B = 16384
D = 64
L = 16
SL = 8

_NC = 2
_NS = 16
_NW = _NC * _NS
_BPW = B // _NW
_C = 32
_NCH = _BPW // _C

_mesh = plsc.VectorSubcoreMesh(core_axis_name="c", subcore_axis_name="s")


@functools.partial(
    pl.kernel,
    out_type=jax.ShapeDtypeStruct((B,), jnp.float32),
    mesh=_mesh,
    scratch_types=[
        pltpu.VMEM((_BPW,), jnp.int32),
        pltpu.VMEM((_BPW,), jnp.int32),
        pltpu.VMEM((2, _C, D), jnp.float32),
        pltpu.VMEM((2, _C, D), jnp.float32),
        pltpu.VMEM((_BPW,), jnp.float32),
        pltpu.VMEM((_BPW,), jnp.float32),
        pltpu.VMEM((L,), jnp.float32),
        pltpu.VMEM((_BPW,), jnp.float32),
        pltpu.VMEM((L * (L + 1),), jnp.float32),
        pltpu.SemaphoreType.DMA,
        pltpu.SemaphoreType.DMA,
        pltpu.SemaphoreType.DMA,
    ],
    compiler_params=pltpu.CompilerParams(needs_layout_passes=False),
)
def _mf_kernel(u_id, i_id, user_emb, user_bias, item_emb, item_bias, mean,
               out, u_idx_v, i_idx_v,
               u_blk, i_blk, bu_v, bi_v, mean_v, out_v, tr_v,
               sem0, sem1, semb):
  wid = lax.axis_index("s") * _NC + lax.axis_index("c")
  base = wid * _BPW
  sems = (sem0, sem1)

  pltpu.sync_copy(u_id.at[pl.ds(base, _BPW)], u_idx_v)
  pltpu.sync_copy(i_id.at[pl.ds(base, _BPW)], i_idx_v)

  cb1 = pltpu.async_copy(user_bias.at[u_idx_v], bu_v, semb)
  cb2 = pltpu.async_copy(item_bias.at[i_idx_v], bi_v, semb)

  pltpu.sync_copy(mean, mean_v)

  def fire(ch, slot):
    for g in range(_C // L):
      uvec = u_idx_v[pl.ds(ch * _C + g * L, L)]
      ivec = i_idx_v[pl.ds(ch * _C + g * L, L)]
      for r in range(L):
        pltpu.async_copy(user_emb.at[uvec[r] >> 3, uvec[r] & 7],
                         u_blk.at[slot, g * L + r], sems[slot])
        pltpu.async_copy(item_emb.at[ivec[r] >> 3, ivec[r] & 7],
                         i_blk.at[slot, g * L + r], sems[slot])

  def drain(slot):
    for h in range(_C // SL):
      pltpu.make_async_copy(user_emb.at[0],
                            u_blk.at[slot, pl.ds(h * SL, SL)],
                            sems[slot]).wait()
      pltpu.make_async_copy(item_emb.at[0],
                            i_blk.at[slot, pl.ds(h * SL, SL)],
                            sems[slot]).wait()

  fire(0, 0)
  cb1.wait()
  cb2.wait()
  mvec = mean_v[...]
  lane17 = lax.iota(jnp.int32, L) * (L + 1)

  def compute(ch, slot):
    for g in range(_C // L):
      uvec = u_idx_v[pl.ds(ch * _C + g * L, L)]
      ivec = i_idx_v[pl.ds(ch * _C + g * L, L)]
      for r in range(L):
        acc = (u_blk[slot, g * L + r, pl.ds(0, L)] *
               i_blk[slot, g * L + r, pl.ds(0, L)])
        for j in range(1, D // L):
          acc = acc + (u_blk[slot, g * L + r, pl.ds(j * L, L)] *
                       i_blk[slot, g * L + r, pl.ds(j * L, L)])
        tr_v[pl.ds(r * (L + 1), L)] = acc
      dots = mvec
      for c in range(L):
        dots = dots + plsc.load_gather(tr_v, [lane17 + c])
      sl = pl.ds(ch * _C + g * L, L)
      out_v[sl] = dots + bu_v[sl] + bi_v[sl]

  def step(t, carry):
    ch0 = t * 2
    fire(ch0 + 1, 1)
    drain(0)
    compute(ch0, 0)

    @pl.when(ch0 + 2 < _NCH)
    def _():
      fire(ch0 + 2, 0)

    drain(1)
    compute(ch0 + 1, 1)
    return carry

  lax.fori_loop(0, _NCH // 2, step, 0)

  pltpu.sync_copy(out_v, out.at[pl.ds(base, _BPW)])


def kernel(u_id, i_id, user_emb, user_bias, item_emb, item_bias, mean):
  u_id = u_id.astype(jnp.int32)
  i_id = i_id.astype(jnp.int32)
  mean16 = jnp.broadcast_to(mean, (L,))
  u3 = jnp.reshape(user_emb, (user_emb.shape[0] // SL, SL, D))
  i3 = jnp.reshape(item_emb, (item_emb.shape[0] // SL, SL, D))
  return _mf_kernel(u_id, i_id, u3, jnp.reshape(user_bias, (-1,)),
                    i3, jnp.reshape(item_bias, (-1,)), mean16)


# speedup vs baseline: 1.0058x; 1.0058x over previous
import functools

import jax
import jax.numpy as jnp
from jax import lax
from jax.experimental import pallas as pl
from jax.experimental.pallas import tpu as pltpu
from jax.experimental.pallas import tpu_sc as plsc

B = 16384
D = 64
L = 16
SL = 8

_NC = 2
_NS = 16
_NW = _NC * _NS
_BPW = B // _NW
_C = 16
_NCH = _BPW // _C

_mesh = plsc.VectorSubcoreMesh(core_axis_name="c", subcore_axis_name="s")


@functools.partial(
    pl.kernel,
    out_type=jax.ShapeDtypeStruct((B,), jnp.float32),
    mesh=_mesh,
    scratch_types=[
        pltpu.VMEM((_BPW,), jnp.int32),
        pltpu.VMEM((_BPW,), jnp.int32),
        pltpu.VMEM((2, _C, D), jnp.float32),
        pltpu.VMEM((2, _C, D), jnp.float32),
        pltpu.VMEM((_BPW,), jnp.float32),
        pltpu.VMEM((_BPW,), jnp.float32),
        pltpu.VMEM((L,), jnp.float32),
        pltpu.VMEM((_BPW,), jnp.float32),
        pltpu.VMEM((L * (L + 1),), jnp.float32),
        pltpu.SemaphoreType.DMA,
        pltpu.SemaphoreType.DMA,
        pltpu.SemaphoreType.DMA,
    ],
    compiler_params=pltpu.CompilerParams(needs_layout_passes=False),
)
def _mf_kernel(u_id, i_id, user_emb, user_bias, item_emb, item_bias, mean,
               out, u_idx_v, i_idx_v,
               u_blk, i_blk, bu_v, bi_v, mean_v, out_v, tr_v,
               sem0, sem1, semb):
  wid = lax.axis_index("s") * _NC + lax.axis_index("c")
  base = wid * _BPW
  sems = (sem0, sem1)

  pltpu.sync_copy(u_id.at[pl.ds(base, _BPW)], u_idx_v)
  pltpu.sync_copy(i_id.at[pl.ds(base, _BPW)], i_idx_v)

  cb1 = pltpu.async_copy(user_bias.at[u_idx_v], bu_v, semb)
  cb2 = pltpu.async_copy(item_bias.at[i_idx_v], bi_v, semb)

  pltpu.sync_copy(mean, mean_v)

  def fire(ch, slot):
    for g in range(_C // L):
      uvec = u_idx_v[pl.ds(ch * _C + g * L, L)]
      ivec = i_idx_v[pl.ds(ch * _C + g * L, L)]
      for r in range(L):
        pltpu.async_copy(user_emb.at[uvec[r] >> 3, uvec[r] & 7],
                         u_blk.at[slot, g * L + r], sems[slot])
        pltpu.async_copy(item_emb.at[ivec[r] >> 3, ivec[r] & 7],
                         i_blk.at[slot, g * L + r], sems[slot])

  def drain(slot):
    for h in range(_C // SL):
      pltpu.make_async_copy(user_emb.at[0],
                            u_blk.at[slot, pl.ds(h * SL, SL)],
                            sems[slot]).wait()
      pltpu.make_async_copy(item_emb.at[0],
                            i_blk.at[slot, pl.ds(h * SL, SL)],
                            sems[slot]).wait()

  fire(0, 0)
  cb1.wait()
  cb2.wait()
  mvec = mean_v[...]
  lane17 = lax.iota(jnp.int32, L) * (L + 1)

  def compute(ch, slot):
    for g in range(_C // L):
      uvec = u_idx_v[pl.ds(ch * _C + g * L, L)]
      ivec = i_idx_v[pl.ds(ch * _C + g * L, L)]
      for r in range(L):
        acc = (u_blk[slot, g * L + r, pl.ds(0, L)] *
               i_blk[slot, g * L + r, pl.ds(0, L)])
        for j in range(1, D // L):
          acc = acc + (u_blk[slot, g * L + r, pl.ds(j * L, L)] *
                       i_blk[slot, g * L + r, pl.ds(j * L, L)])
        tr_v[pl.ds(r * (L + 1), L)] = acc
      dots = mvec
      for c in range(L):
        dots = dots + plsc.load_gather(tr_v, [lane17 + c])
      sl = pl.ds(ch * _C + g * L, L)
      out_v[sl] = dots + bu_v[sl] + bi_v[sl]

  def step(t, carry):
    ch0 = t * 2
    fire(ch0 + 1, 1)
    drain(0)
    compute(ch0, 0)

    @pl.when(ch0 + 2 < _NCH)
    def _():
      fire(ch0 + 2, 0)

    drain(1)
    compute(ch0 + 1, 1)
    return carry

  lax.fori_loop(0, _NCH // 2, step, 0)

  pltpu.sync_copy(out_v, out.at[pl.ds(base, _BPW)])


def kernel(u_id, i_id, user_emb, user_bias, item_emb, item_bias, mean):
  u_id = u_id.astype(jnp.int32)
  i_id = i_id.astype(jnp.int32)
  mean16 = jnp.broadcast_to(mean, (L,))
  u3 = jnp.reshape(user_emb, (user_emb.shape[0] // SL, SL, D))
  i3 = jnp.reshape(item_emb, (item_emb.shape[0] // SL, SL, D))
  return _mf_kernel(u_id, i_id, u3, jnp.reshape(user_bias, (-1,)),
                    i3, jnp.reshape(item_bias, (-1,)), mean16)
